# Initial kernel scaffold; baseline (speedup 1.0000x reference)
import jax
import jax.numpy as jnp
from jax.experimental import pallas as pl


def kernel(x, edge_index, Wl1, bl1, Wr1, br1, att1, bias1, g1, be1, Wl2, bl2, Wr2, br2, att2, bias2, g2, be2, Wc, bc):
    raise NotImplementedError("write your pallas kernel here")


# speedup vs baseline: 1.0166x; 1.0166x over previous
import functools

import jax
import jax.numpy as jnp
from jax.experimental import pallas as pl
from jax.experimental.pallas import tpu as pltpu


def _proj_body(x_ref, wl_ref, bl_ref, wr_ref, br_ref, xl_ref, xr_ref):
    x = x_ref[...]
    xl_ref[...] = jnp.dot(x, wl_ref[...], preferred_element_type=jnp.float32) + bl_ref[...]
    xr_ref[...] = jnp.dot(x, wr_ref[...], preferred_element_type=jnp.float32) + br_ref[...]


def _dense_proj(x, Wl, bl, Wr, br, block=2000):
    n, d = x.shape
    f = Wl.shape[1]
    bl2 = bl.reshape(1, f)
    br2 = br.reshape(1, f)
    grid = n // block
    xl, xr = pl.pallas_call(
        _proj_body,
        grid=(grid,),
        in_specs=[
            pl.BlockSpec((block, d), lambda i: (i, 0)),
            pl.BlockSpec((d, f), lambda i: (0, 0)),
            pl.BlockSpec((1, f), lambda i: (0, 0)),
            pl.BlockSpec((d, f), lambda i: (0, 0)),
            pl.BlockSpec((1, f), lambda i: (0, 0)),
        ],
        out_specs=[
            pl.BlockSpec((block, f), lambda i: (i, 0)),
            pl.BlockSpec((block, f), lambda i: (i, 0)),
        ],
        out_shape=[
            jax.ShapeDtypeStruct((n, f), jnp.float32),
            jax.ShapeDtypeStruct((n, f), jnp.float32),
        ],
    )(x, Wl, bl2, Wr, br2)
    return xl, xr


def _gat_layer(x, src, dst, Wl, bl, Wr, br, att, bias, heads, ch):
    n = x.shape[0]
    xl, xr = _dense_proj(x, Wl, bl, Wr, br)
    xl = xl.reshape(n, heads, ch)
    xr = xr.reshape(n, heads, ch)
    e = jax.nn.leaky_relu(xl[src] + xr[dst], negative_slope=0.2)
    alpha = (e * att).sum(-1)
    amax = jax.ops.segment_max(alpha, dst, num_segments=n)
    amax = jnp.where(jnp.isfinite(amax), amax, 0.0)
    ex = jnp.exp(alpha - amax[dst])
    den = jax.ops.segment_sum(ex, dst, num_segments=n)
    a = ex / (den[dst] + 1e-16)
    out = jax.ops.segment_sum(xl[src] * a[..., None], dst, num_segments=n)
    return out.reshape(n, heads * ch) + bias


def _bn_gelu(x, g, b):
    mu = x.mean(0)
    var = x.var(0)
    return jax.nn.gelu(g * (x - mu) * jax.lax.rsqrt(var + 1e-5) + b, approximate=False)


def kernel(x, edge_index, Wl1, bl1, Wr1, br1, att1, bias1, g1, be1, Wl2, bl2, Wr2, br2, att2, bias2, g2, be2, Wc, bc):
    n = x.shape[0]
    loop = jnp.arange(n, dtype=edge_index.dtype)
    src = jnp.concatenate([edge_index[0], loop])
    dst = jnp.concatenate([edge_index[1], loop])
    h = _gat_layer(x, src, dst, Wl1, bl1, Wr1, br1, att1, bias1, 8, 64)
    h = _bn_gelu(h, g1, be1)
    h = _gat_layer(h, src, dst, Wl2, bl2, Wr2, br2, att2, bias2, 1, 64)
    h = _bn_gelu(h, g2, be2)
    return jax.nn.log_softmax(h @ Wc + bc, axis=1)


# speedup vs baseline: 5.1573x; 5.0729x over previous
import functools

import jax
import jax.numpy as jnp
from jax import lax
from jax.experimental import pallas as pl
from jax.experimental.pallas import tpu as pltpu
from jax.experimental.pallas import tpu_sc as plsc

_NC = 2
_NS = 16
_NW = _NC * _NS
_GRAIN = 1536


def _mesh():
    return plsc.VectorSubcoreMesh(core_axis_name="c", subcore_axis_name="s")


def _proj_body(x_ref, wl_ref, bl_ref, wr_ref, br_ref, xl_ref, xr_ref):
    x = x_ref[...]
    xl_ref[...] = jnp.dot(x, wl_ref[...], preferred_element_type=jnp.float32) + bl_ref[...]
    xr_ref[...] = jnp.dot(x, wr_ref[...], preferred_element_type=jnp.float32) + br_ref[...]


def _dense_proj(x, Wl, bl, Wr, br, block=2000):
    n, d = x.shape
    f = Wl.shape[1]
    grid = n // block
    return pl.pallas_call(
        _proj_body,
        grid=(grid,),
        in_specs=[
            pl.BlockSpec((block, d), lambda i: (i, 0)),
            pl.BlockSpec((d, f), lambda i: (0, 0)),
            pl.BlockSpec((1, f), lambda i: (0, 0)),
            pl.BlockSpec((d, f), lambda i: (0, 0)),
            pl.BlockSpec((1, f), lambda i: (0, 0)),
        ],
        out_specs=[
            pl.BlockSpec((block, f), lambda i: (i, 0)),
            pl.BlockSpec((block, f), lambda i: (i, 0)),
        ],
        out_shape=[
            jax.ShapeDtypeStruct((n, f), jnp.float32),
            jax.ShapeDtypeStruct((n, f), jnp.float32),
        ],
    )(x, Wl, bl.reshape(1, f), Wr, br.reshape(1, f))


def _sc_gather(table, idx, k, pw):
    _, F = table.shape
    e2p = idx.shape[0]
    B = k * 128
    nb = pw // B

    @functools.partial(
        pl.kernel, mesh=_mesh(),
        out_type=jax.ShapeDtypeStruct((e2p, F), jnp.float32),
        scratch_types=[
            pltpu.VMEM((B,), jnp.int32),
            pltpu.VMEM((B, F), jnp.float32),
            pltpu.SemaphoreType.DMA,
        ],
    )
    def kfn(table_hbm, idx_hbm, out_hbm, idx_v, rows_v, sem):
        w = lax.axis_index("s") * _NC + lax.axis_index("c")

        @pl.loop(0, nb)
        def _(i):
            off = w * pw + i * B
            pltpu.sync_copy(idx_hbm.at[pl.ds(off, B)], idx_v)
            if k == 1:
                pltpu.async_copy(table_hbm.at[idx_v], rows_v, sem).wait()
            else:
                cps = [
                    pltpu.async_copy(
                        table_hbm.at[idx_v.at[pl.ds(j * 128, 128)]],
                        rows_v.at[pl.ds(j * 128, 128)],
                        sem,
                    )
                    for j in range(k)
                ]
                for cp in cps:
                    cp.wait()
            pltpu.sync_copy(rows_v, out_hbm.at[pl.ds(off, B)])

    return kfn(table, idx)


def _acc_copy(src, dst, s, nn):
    slab = (nn // _NS) & ~7
    pltpu.sync_copy(src.at[pl.ds(s * slab, slab)], dst.at[pl.ds(s * slab, slab)])
    tail = nn - _NS * slab
    if tail:
        @pl.when(s == 0)
        def _():
            pltpu.sync_copy(src.at[pl.ds(_NS * slab, tail)],
                            dst.at[pl.ds(_NS * slab, tail)])


def _sc_scatter(v, idx, zc, nch, pw):
    nn = zc.shape[0]
    B = 128
    nb = pw // B

    @functools.partial(
        pl.kernel, mesh=_mesh(),
        out_type=jax.ShapeDtypeStruct((_NC * nch, nn, 128), jnp.float32),
        scratch_types=[
            pltpu.VMEM((B,), jnp.int32),
            pltpu.VMEM((B, 128), jnp.float32),
            pltpu.VMEM_SHARED((nn, 128), jnp.float32),
        ],
    )
    def kfn(v_hbm, idx_hbm, zc_hbm, agg_hbm, idx_v, rows_v, acc_sh):
        c = lax.axis_index("c")
        s = lax.axis_index("s")
        w = s * _NC + c
        for ch in range(nch):
            _acc_copy(zc_hbm, acc_sh, s, nn)
            plsc.subcore_barrier()

            @pl.loop(0, nb)
            def _(i):
                off = w * pw + i * B
                pltpu.sync_copy(idx_hbm.at[pl.ds(off, B)], idx_v)
                pltpu.sync_copy(v_hbm.at[pl.ds(off, B), pl.ds(ch * 128, 128)],
                                rows_v)
                pltpu.sync_copy(rows_v, acc_sh.at[idx_v], add=True)

            plsc.subcore_barrier()
            _acc_copy(acc_sh, agg_hbm.at[c * nch + ch], s, nn)

    return kfn(v, idx, zc)


def _gat_layer(x, src, dst, emask, Wl, bl, Wr, br, att, bias, heads, ch):
    n = x.shape[0]
    hc = heads * ch
    e2p = src.shape[0]
    pw = e2p // _NW
    xl, xr = _dense_proj(x, Wl, bl, Wr, br)

    s_self = (jax.nn.leaky_relu(xl + xr, negative_slope=0.2)
              .reshape(n, heads, ch) * att).sum(-1)
    augw = -(hc + heads) % 128 + hc + heads
    xr_aug = jnp.concatenate(
        [xr, s_self, jnp.zeros((n, augw - hc - heads), jnp.float32)], axis=1)
    if hc % 128:
        xl_t = jnp.concatenate(
            [xl, jnp.zeros((n, 128 - hc % 128), jnp.float32)], axis=1)
    else:
        xl_t = xl

    kl = 1 if xl_t.shape[1] > 256 else 4
    kr = 1 if xr_aug.shape[1] > 256 else 4
    xl_src = _sc_gather(xl_t, src, kl, pw)[:, :hc]
    xr_g = _sc_gather(xr_aug, dst, kr, pw)
    xr_dst = xr_g[:, :hc]
    s_dst = xr_g[:, hc:hc + heads]

    u = xl_src + xr_dst
    uh = u.reshape(e2p, heads, ch)
    alpha = (jax.nn.leaky_relu(uh, negative_slope=0.2) * att).sum(-1)
    ex = jnp.exp(alpha - s_dst)
    ex = jnp.where(emask[:, None], ex, 0.0)
    v = (uh * ex[..., None]).reshape(e2p, hc)
    v_aug = jnp.concatenate(
        [v, ex, jnp.zeros((e2p, augw - hc - heads), jnp.float32)], axis=1)

    nch = augw // 128
    zc = jnp.zeros((n, 128), jnp.float32)
    parts = _sc_scatter(v_aug, dst, zc, nch, pw)

    agg = (parts[:nch] + parts[nch:]).transpose(1, 0, 2).reshape(n, nch * 128)
    aggu = agg[:, :hc]
    den = agg[:, hc:hc + heads]
    out = (aggu.reshape(n, heads, ch) / den[..., None]
           - xr.reshape(n, heads, ch)).reshape(n, hc)
    return out + bias


def _bn_gelu(x, g, b):
    mu = x.mean(0)
    var = x.var(0)
    return jax.nn.gelu(g * (x - mu) * jax.lax.rsqrt(var + 1e-5) + b,
                       approximate=False)


def kernel(x, edge_index, Wl1, bl1, Wr1, br1, att1, bias1, g1, be1,
           Wl2, bl2, Wr2, br2, att2, bias2, g2, be2, Wc, bc):
    n = x.shape[0]
    e = edge_index.shape[1]
    e2 = e + n
    grain = _NW * _GRAIN
    e2p = ((e2 + grain - 1) // grain) * grain

    loop = jnp.arange(n, dtype=edge_index.dtype)
    pad = e2p - e2
    src = jnp.pad(jnp.concatenate([edge_index[0], loop]), (0, pad))
    dst = jnp.pad(jnp.concatenate([edge_index[1], loop]), (0, pad))
    emask = jnp.arange(e2p, dtype=jnp.int32) < e2

    h = _gat_layer(x, src, dst, emask, Wl1, bl1, Wr1, br1, att1, bias1, 8, 64)
    h = _bn_gelu(h, g1, be1)
    h = _gat_layer(h, src, dst, emask, Wl2, bl2, Wr2, br2, att2, bias2, 1, 64)
    h = _bn_gelu(h, g2, be2)
    return jax.nn.log_softmax(h @ Wc + bc, axis=1)


# speedup vs baseline: 5.9696x; 1.1575x over previous
import functools

import jax
import jax.numpy as jnp
from jax import lax
from jax.experimental import pallas as pl
from jax.experimental.pallas import tpu as pltpu
from jax.experimental.pallas import tpu_sc as plsc

_NC = 2
_NS = 16
_NW = _NC * _NS
_GRAIN = 1024


def _mesh():
    return plsc.VectorSubcoreMesh(core_axis_name="c", subcore_axis_name="s")


def _proj_l1_body(x_ref, wl_ref, bl_ref, wr_ref, br_ref, a2_ref,
                  xl_ref, xr_ref):
    x = x_ref[...]
    xl = jnp.dot(x, wl_ref[...], preferred_element_type=jnp.float32) + bl_ref[...]
    xr = jnp.dot(x, wr_ref[...], preferred_element_type=jnp.float32) + br_ref[...]
    t = xl + xr
    lk = jnp.where(t >= 0, t, 0.2 * t)
    xl_ref[...] = xl
    xr_ref[:, :512] = xr
    xr_ref[:, 512:640] = jnp.dot(lk, a2_ref[...],
                                 preferred_element_type=jnp.float32)


def _proj_l2_body(x_ref, wl_ref, bl_ref, wr_ref, br_ref, a2_ref, rp_ref,
                  xl_ref, xr_ref):
    x = x_ref[...]
    xl = jnp.dot(x, wl_ref[...], preferred_element_type=jnp.float32) + bl_ref[...]
    xr = jnp.dot(x, wr_ref[...], preferred_element_type=jnp.float32) + br_ref[...]
    t = xl + xr
    lk = jnp.where(t >= 0, t, 0.2 * t)
    rp = rp_ref[...]
    xl_ref[...] = jnp.dot(xl, rp, preferred_element_type=jnp.float32)
    xr_ref[...] = (jnp.dot(xr, rp, preferred_element_type=jnp.float32)
                   + jnp.dot(lk, a2_ref[...],
                             preferred_element_type=jnp.float32))


def _dense_proj(x, Wl, bl, Wr, br, a2, rpad, xw, w, block=2000):
    n, d = x.shape
    hc = Wl.shape[1]
    grid = n // block
    body = _proj_l1_body if rpad is None else _proj_l2_body
    in_specs = [
        pl.BlockSpec((block, d), lambda i: (i, 0)),
        pl.BlockSpec((d, hc), lambda i: (0, 0)),
        pl.BlockSpec((1, hc), lambda i: (0, 0)),
        pl.BlockSpec((d, hc), lambda i: (0, 0)),
        pl.BlockSpec((1, hc), lambda i: (0, 0)),
        pl.BlockSpec((hc, 128), lambda i: (0, 0)),
    ]
    args = [x, Wl, bl.reshape(1, hc), Wr, br.reshape(1, hc), a2]
    if rpad is not None:
        in_specs.append(pl.BlockSpec((hc, xw), lambda i: (0, 0)))
        args.append(rpad)
    return pl.pallas_call(
        body,
        grid=(grid,),
        in_specs=in_specs,
        out_specs=[
            pl.BlockSpec((block, xw), lambda i: (i, 0)),
            pl.BlockSpec((block, w), lambda i: (i, 0)),
        ],
        out_shape=[
            jax.ShapeDtypeStruct((n, xw), jnp.float32),
            jax.ShapeDtypeStruct((n, w), jnp.float32),
        ],
    )(*args)


def _edge_l1_body(e2, b2, xl_ref, xr_ref, a2_ref, p_ref, p2_ref, out_ref):
    xl = xl_ref[...]
    xr = xr_ref[:, :512]
    s = xr_ref[:, 512:520]
    u = xl + xr
    lk = jnp.where(u >= 0, u, 0.2 * u)
    alpha = jnp.dot(lk, a2_ref[...], preferred_element_type=jnp.float32)[:, :8]
    row = pl.program_id(0) * b2 + lax.broadcasted_iota(jnp.int32, (b2, 1), 0)
    aex = jnp.where(row < e2, jnp.exp(alpha - s), 0.0)
    out_ref[:, :512] = u * jnp.dot(aex, p_ref[...],
                                   preferred_element_type=jnp.float32)
    out_ref[:, 512:640] = jnp.dot(aex, p2_ref[...],
                                  preferred_element_type=jnp.float32)


def _edge_l2_body(e2, b2, xl_ref, xr_ref, a2_ref, p_ref, p2_ref, out_ref):
    u = xl_ref[...] + xr_ref[...]
    lk0 = u[:, :64]
    lk = jnp.where(lk0 >= 0, lk0, 0.2 * lk0)
    alpha = jnp.dot(lk, a2_ref[...], preferred_element_type=jnp.float32)[:, :1]
    s = xr_ref[:, 64:65]
    row = pl.program_id(0) * b2 + lax.broadcasted_iota(jnp.int32, (b2, 1), 0)
    aex = jnp.where(row < e2, jnp.exp(alpha - s), 0.0)
    out_ref[...] = (u * jnp.dot(aex, p_ref[...],
                                preferred_element_type=jnp.float32)
                    + jnp.dot(aex, p2_ref[...],
                              preferred_element_type=jnp.float32))


def _edge_math(xl_src, xr_g, a2, p, p2, e2, heads, b2=2048):
    e2p, xw = xl_src.shape
    w = xr_g.shape[1]
    grid = e2p // b2
    body = functools.partial(
        _edge_l1_body if heads == 8 else _edge_l2_body, e2, b2)
    return pl.pallas_call(
        body,
        grid=(grid,),
        in_specs=[
            pl.BlockSpec((b2, xw), lambda i: (i, 0)),
            pl.BlockSpec((b2, w), lambda i: (i, 0)),
            pl.BlockSpec(a2.shape, lambda i: (0, 0)),
            pl.BlockSpec(p.shape, lambda i: (0, 0)),
            pl.BlockSpec(p2.shape, lambda i: (0, 0)),
        ],
        out_specs=pl.BlockSpec((b2, w), lambda i: (i, 0)),
        out_shape=jax.ShapeDtypeStruct((e2p, w), jnp.float32),
    )(xl_src, xr_g, a2, p, p2)


def _sc_gather(table, idx, B, pw):
    _, F = table.shape
    e2p = idx.shape[0]
    nb = pw // B

    @functools.partial(
        pl.kernel, mesh=_mesh(),
        out_type=jax.ShapeDtypeStruct((e2p, F), jnp.float32),
        scratch_types=[
            pltpu.VMEM((pw,), jnp.int32),
            pltpu.VMEM((B, F), jnp.float32),
            pltpu.VMEM((B, F), jnp.float32),
            pltpu.SemaphoreType.DMA,
            pltpu.SemaphoreType.DMA,
        ],
    )
    def kfn(table_hbm, idx_hbm, out_hbm, idx_all, rows0, rows1, sem0, sem1):
        w = lax.axis_index("s") * _NC + lax.axis_index("c")
        base = w * pw
        pltpu.sync_copy(idx_hbm.at[pl.ds(base, pw)], idx_all)

        def gstart(i, rows, sem):
            pltpu.async_copy(table_hbm.at[idx_all.at[pl.ds(i * B, B)]],
                             rows, sem)

        def gwait(rows, sem):
            pltpu.make_async_copy(table_hbm.at[pl.ds(0, B)], rows, sem).wait()

        def put(i, rows):
            pltpu.sync_copy(rows, out_hbm.at[pl.ds(base + i * B, B)])

        gstart(0, rows0, sem0)

        @pl.loop(0, nb // 2 - 1)
        def _(t):
            i0 = 2 * t
            gstart(i0 + 1, rows1, sem1)
            gwait(rows0, sem0)
            put(i0, rows0)
            gstart(i0 + 2, rows0, sem0)
            gwait(rows1, sem1)
            put(i0 + 1, rows1)

        gstart(nb - 1, rows1, sem1)
        gwait(rows0, sem0)
        put(nb - 2, rows0)
        gwait(rows1, sem1)
        put(nb - 1, rows1)

    return kfn(table, idx)


def _acc_copy(src, dst, s, nn):
    slab = (nn // _NS) & ~7
    pltpu.sync_copy(src.at[pl.ds(s * slab, slab)], dst.at[pl.ds(s * slab, slab)])
    tail = nn - _NS * slab
    if tail:
        @pl.when(s == 0)
        def _():
            pltpu.sync_copy(src.at[pl.ds(_NS * slab, tail)],
                            dst.at[pl.ds(_NS * slab, tail)])


def _sc_scatter(v, idx2, zc, nch, pw):
    nn = zc.shape[0]
    B = 128
    nb = pw // B
    irw = pw // 128

    @functools.partial(
        pl.kernel, mesh=_mesh(),
        out_type=jax.ShapeDtypeStruct((_NC * nch, nn, 128), jnp.float32),
        scratch_types=[
            pltpu.VMEM((irw, 128), jnp.int32),
            pltpu.VMEM((B, 128), jnp.float32),
            pltpu.VMEM((B, 128), jnp.float32),
            pltpu.VMEM_SHARED((nn, 128), jnp.float32),
            pltpu.SemaphoreType.DMA,
            pltpu.SemaphoreType.DMA,
        ],
    )
    def kfn(v_hbm, idx_hbm, zc_hbm, agg_hbm,
            idx_all, rows0, rows1, acc_sh, sem0, sem1):
        c = lax.axis_index("c")
        s = lax.axis_index("s")
        w = s * _NC + c
        base = w * pw
        pltpu.sync_copy(idx_hbm.at[pl.ds(w * irw, irw)], idx_all)

        for ch in range(nch):
            _acc_copy(zc_hbm, acc_sh, s, nn)
            plsc.subcore_barrier()

            def lstart(i, rows, sem):
                pltpu.async_copy(
                    v_hbm.at[pl.ds(base + i * B, B), pl.ds(ch * 128, 128)],
                    rows, sem)

            def lwait(rows, sem):
                pltpu.make_async_copy(
                    v_hbm.at[pl.ds(base, B), pl.ds(ch * 128, 128)],
                    rows, sem).wait()

            def scat(i, rows):
                pltpu.sync_copy(rows, acc_sh.at[idx_all.at[i]], add=True)

            lstart(0, rows0, sem0)

            @pl.loop(0, nb // 2 - 1)
            def _(t):
                i0 = 2 * t
                lstart(i0 + 1, rows1, sem1)
                lwait(rows0, sem0)
                scat(i0, rows0)
                lstart(i0 + 2, rows0, sem0)
                lwait(rows1, sem1)
                scat(i0 + 1, rows1)

            lstart(nb - 1, rows1, sem1)
            lwait(rows0, sem0)
            scat(nb - 2, rows0)
            lwait(rows1, sem1)
            scat(nb - 1, rows1)

            plsc.subcore_barrier()
            _acc_copy(acc_sh, agg_hbm.at[c * nch + ch], s, nn)

    return kfn(v, idx2, zc)


def _gat_layer(x, src, dst, dst2, e2, Wl, bl, Wr, br, att, bias, heads, ch):
    n = x.shape[0]
    hc = heads * ch
    e2p = src.shape[0]
    pw = e2p // _NW
    augw = -(hc + heads) % 128 + hc + heads
    xw = -hc % 128 + hc

    af = att.reshape(hc)
    hrep = jnp.repeat(jnp.arange(heads), ch)
    if heads == 8:
        a2_proj = jnp.zeros((hc, 128), jnp.float32).at[
            jnp.arange(hc), hrep].set(af)
        a2_edge = a2_proj
        p = jnp.repeat(jnp.eye(heads, dtype=jnp.float32), ch, axis=1)
        p2 = jnp.eye(heads, 128, dtype=jnp.float32)
        rpad = None
    else:
        a2_proj = jnp.zeros((hc, 128), jnp.float32).at[
            jnp.arange(hc), 64].set(af)
        a2_edge = jnp.zeros((hc, 128), jnp.float32).at[
            jnp.arange(hc), 0].set(af)
        p = (jnp.arange(128) < 64).astype(jnp.float32).reshape(1, 128)
        p2 = jnp.zeros((1, 128), jnp.float32).at[0, 64].set(1.0)
        rpad = jnp.eye(hc, 128, dtype=jnp.float32)

    xl_t, xr_aug = _dense_proj(x, Wl, bl, Wr, br, a2_proj, rpad, xw, augw)

    bg = 64 if augw > 256 else 128
    xl_src = _sc_gather(xl_t, src, bg, pw)
    xr_g = _sc_gather(xr_aug, dst, bg, pw)

    v_aug = _edge_math(xl_src, xr_g, a2_edge, p, p2, e2, heads)

    nch = augw // 128
    zc = jnp.zeros((n, 128), jnp.float32)
    parts = _sc_scatter(v_aug, dst2, zc, nch, pw)

    agg = (parts[:nch] + parts[nch:]).transpose(1, 0, 2).reshape(n, nch * 128)
    aggu = agg[:, :hc]
    den = agg[:, hc:hc + heads]
    xr = xr_aug[:, :hc]
    out = (aggu.reshape(n, heads, ch) / den[..., None]
           - xr.reshape(n, heads, ch)).reshape(n, hc)
    return out + bias


def _bn_gelu(x, g, b):
    mu = x.mean(0)
    var = x.var(0)
    return jax.nn.gelu(g * (x - mu) * jax.lax.rsqrt(var + 1e-5) + b,
                       approximate=False)


def kernel(x, edge_index, Wl1, bl1, Wr1, br1, att1, bias1, g1, be1,
           Wl2, bl2, Wr2, br2, att2, bias2, g2, be2, Wc, bc):
    n = x.shape[0]
    e = edge_index.shape[1]
    e2 = e + n
    grain = _NW * _GRAIN
    e2p = ((e2 + grain - 1) // grain) * grain

    loop = jnp.arange(n, dtype=edge_index.dtype)
    pad = e2p - e2
    src = jnp.pad(jnp.concatenate([edge_index[0], loop]), (0, pad))
    dst = jnp.pad(jnp.concatenate([edge_index[1], loop]), (0, pad))
    dst2 = dst.reshape(e2p // 128, 128)

    h = _gat_layer(x, src, dst, dst2, e2, Wl1, bl1, Wr1, br1, att1, bias1, 8, 64)
    h = _bn_gelu(h, g1, be1)
    h = _gat_layer(h, src, dst, dst2, e2, Wl2, bl2, Wr2, br2, att2, bias2, 1, 64)
    h = _bn_gelu(h, g2, be2)
    return jax.nn.log_softmax(h @ Wc + bc, axis=1)


# speedup vs baseline: 6.3490x; 1.0636x over previous
import functools

import jax
import jax.numpy as jnp
from jax import lax
from jax.experimental import pallas as pl
from jax.experimental.pallas import tpu as pltpu
from jax.experimental.pallas import tpu_sc as plsc

_NC = 2
_NS = 16
_NW = _NC * _NS
_GRAIN = 1024


def _mesh():
    return plsc.VectorSubcoreMesh(core_axis_name="c", subcore_axis_name="s")


def _pack2(a, b):
    hi = jnp.uint32(0xFFFF0000)
    aw = lax.bitcast_convert_type(a, jnp.uint32)
    bw = lax.bitcast_convert_type(b, jnp.uint32)
    return lax.bitcast_convert_type((bw & hi) | (aw >> 16), jnp.float32)


def _unpack_lo(w):
    ww = lax.bitcast_convert_type(w, jnp.uint32)
    return lax.bitcast_convert_type(ww << 16, jnp.float32)


def _unpack_hi(w):
    ww = lax.bitcast_convert_type(w, jnp.uint32)
    return lax.bitcast_convert_type(ww & jnp.uint32(0xFFFF0000), jnp.float32)


def _proj_l1_body(x_ref, wl_ref, bl_ref, wr_ref, br_ref, a2_ref,
                  xl_ref, xr_ref, xrf_ref):
    x = x_ref[...]
    xl = jnp.dot(x, wl_ref[...], preferred_element_type=jnp.float32) + bl_ref[...]
    xr = jnp.dot(x, wr_ref[...], preferred_element_type=jnp.float32) + br_ref[...]
    t = xl + xr
    lk = jnp.where(t >= 0, t, 0.2 * t)
    s128 = jnp.dot(lk, a2_ref[...], preferred_element_type=jnp.float32)
    xl_ref[...] = _pack2(xl[:, :256], xl[:, 256:])
    xr_ref[:, :256] = _pack2(xr[:, :256], xr[:, 256:])
    xr_ref[:, 256:384] = _pack2(s128, jnp.zeros_like(s128))
    xrf_ref[...] = xr


def _proj_l2_body(x_ref, wl_ref, bl_ref, wr_ref, br_ref, a2_ref, rp_ref,
                  xl_ref, xr_ref, xrf_ref):
    x = x_ref[...]
    xl = jnp.dot(x, wl_ref[...], preferred_element_type=jnp.float32) + bl_ref[...]
    xr = jnp.dot(x, wr_ref[...], preferred_element_type=jnp.float32) + br_ref[...]
    t = xl + xr
    lk = jnp.where(t >= 0, t, 0.2 * t)
    rp = rp_ref[...]
    xl_ref[...] = jnp.dot(xl, rp, preferred_element_type=jnp.float32)
    xr_ref[...] = (jnp.dot(xr, rp, preferred_element_type=jnp.float32)
                   + jnp.dot(lk, a2_ref[...],
                             preferred_element_type=jnp.float32))
    xrf_ref[...] = xr


def _dense_proj(x, Wl, bl, Wr, br, a2, rpad, xw, w, block=2000):
    n, d = x.shape
    hc = Wl.shape[1]
    grid = n // block
    body = _proj_l1_body if rpad is None else _proj_l2_body
    in_specs = [
        pl.BlockSpec((block, d), lambda i: (i, 0)),
        pl.BlockSpec((d, hc), lambda i: (0, 0)),
        pl.BlockSpec((1, hc), lambda i: (0, 0)),
        pl.BlockSpec((d, hc), lambda i: (0, 0)),
        pl.BlockSpec((1, hc), lambda i: (0, 0)),
        pl.BlockSpec((hc, 128), lambda i: (0, 0)),
    ]
    args = [x, Wl, bl.reshape(1, hc), Wr, br.reshape(1, hc), a2]
    if rpad is not None:
        in_specs.append(pl.BlockSpec((hc, xw), lambda i: (0, 0)))
        args.append(rpad)
    return pl.pallas_call(
        body,
        grid=(grid,),
        in_specs=in_specs,
        out_specs=[
            pl.BlockSpec((block, xw), lambda i: (i, 0)),
            pl.BlockSpec((block, w), lambda i: (i, 0)),
            pl.BlockSpec((block, hc), lambda i: (i, 0)),
        ],
        out_shape=[
            jax.ShapeDtypeStruct((n, xw), jnp.float32),
            jax.ShapeDtypeStruct((n, w), jnp.float32),
            jax.ShapeDtypeStruct((n, hc), jnp.float32),
        ],
    )(*args)


def _edge_l1_body(e2, b2, xl_ref, xr_ref, a2a_ref, a2b_ref, pa_ref, pb_ref,
                  p2_ref, out_ref):
    xlw = xl_ref[...]
    xrw = xr_ref[:, :256]
    ua = _unpack_lo(xlw) + _unpack_lo(xrw)
    ub = _unpack_hi(xlw) + _unpack_hi(xrw)
    s = _unpack_lo(xr_ref[:, 256:384])[:, :8]
    lka = jnp.where(ua >= 0, ua, 0.2 * ua)
    lkb = jnp.where(ub >= 0, ub, 0.2 * ub)
    alpha = (jnp.dot(lka, a2a_ref[...], preferred_element_type=jnp.float32)
             + jnp.dot(lkb, a2b_ref[...],
                       preferred_element_type=jnp.float32))[:, :8]
    row = pl.program_id(0) * b2 + lax.broadcasted_iota(jnp.int32, (b2, 1), 0)
    aex = jnp.where(row < e2, jnp.exp(alpha - s), 0.0)
    out_ref[:, :256] = ua * jnp.dot(aex, pa_ref[...],
                                    preferred_element_type=jnp.float32)
    out_ref[:, 256:512] = ub * jnp.dot(aex, pb_ref[...],
                                       preferred_element_type=jnp.float32)
    out_ref[:, 512:640] = jnp.dot(aex, p2_ref[...],
                                  preferred_element_type=jnp.float32)


def _edge_l2_body(e2, b2, xl_ref, xr_ref, a2_ref, p_ref, p2_ref, out_ref):
    u = xl_ref[...] + xr_ref[...]
    lk0 = u[:, :64]
    lk = jnp.where(lk0 >= 0, lk0, 0.2 * lk0)
    alpha = jnp.dot(lk, a2_ref[...], preferred_element_type=jnp.float32)[:, :1]
    s = xr_ref[:, 64:65]
    row = pl.program_id(0) * b2 + lax.broadcasted_iota(jnp.int32, (b2, 1), 0)
    aex = jnp.where(row < e2, jnp.exp(alpha - s), 0.0)
    out_ref[...] = (u * jnp.dot(aex, p_ref[...],
                                preferred_element_type=jnp.float32)
                    + jnp.dot(aex, p2_ref[...],
                              preferred_element_type=jnp.float32))


def _edge_math(xl_src, xr_g, mats, e2, heads, outw, b2=2048):
    e2p, xw = xl_src.shape
    w = xr_g.shape[1]
    grid = e2p // b2
    body = functools.partial(
        _edge_l1_body if heads == 8 else _edge_l2_body, e2, b2)
    return pl.pallas_call(
        body,
        grid=(grid,),
        in_specs=[
            pl.BlockSpec((b2, xw), lambda i: (i, 0)),
            pl.BlockSpec((b2, w), lambda i: (i, 0)),
        ] + [pl.BlockSpec(m.shape, lambda i: (0, 0)) for m in mats],
        out_specs=pl.BlockSpec((b2, outw), lambda i: (i, 0)),
        out_shape=jax.ShapeDtypeStruct((e2p, outw), jnp.float32),
    )(xl_src, xr_g, *mats)


def _sc_gather(table, idx, B, pw):
    _, F = table.shape
    e2p = idx.shape[0]
    nb = pw // B

    @functools.partial(
        pl.kernel, mesh=_mesh(),
        out_type=jax.ShapeDtypeStruct((e2p, F), table.dtype),
        scratch_types=(
            [pltpu.VMEM((pw,), jnp.int32)]
            + [pltpu.VMEM((B, F), table.dtype) for _ in range(4)]
            + [pltpu.SemaphoreType.DMA for _ in range(8)]
        ),
    )
    def kfn(table_hbm, idx_hbm, out_hbm, idx_all,
            r0, r1, r2, r3, g0, g1, g2, g3, w0, w1, w2, w3):
        rows = [r0, r1, r2, r3]
        gsem = [g0, g1, g2, g3]
        wsem = [w0, w1, w2, w3]
        w = lax.axis_index("s") * _NC + lax.axis_index("c")
        base = w * pw
        pltpu.sync_copy(idx_hbm.at[pl.ds(base, pw)], idx_all)

        def gstart(i, sl):
            pltpu.async_copy(table_hbm.at[idx_all.at[pl.ds(i * B, B)]],
                             rows[sl], gsem[sl])

        def gwait(sl):
            pltpu.make_async_copy(table_hbm.at[pl.ds(0, B)],
                                  rows[sl], gsem[sl]).wait()

        def wstart(i, sl):
            pltpu.async_copy(rows[sl], out_hbm.at[pl.ds(base + i * B, B)],
                             wsem[sl])

        def wwait(sl):
            pltpu.make_async_copy(table_hbm.at[pl.ds(0, B)],
                                  rows[sl], wsem[sl]).wait()

        for sl in range(4):
            gstart(sl, sl)

        @pl.loop(0, nb // 4 - 1)
        def _(t):
            i0 = 4 * t
            for sl in range(4):
                gwait(sl)
                wstart(i0 + sl, sl)
            for sl in range(4):
                wwait(sl)
                gstart(i0 + 4 + sl, sl)

        i0 = nb - 4
        for sl in range(4):
            gwait(sl)
            wstart(i0 + sl, sl)
        for sl in range(4):
            wwait(sl)

    return kfn(table, idx)


def _acc_copy(src, dst, s, nn):
    slab = (nn // _NS) & ~7
    pltpu.sync_copy(src.at[pl.ds(s * slab, slab)], dst.at[pl.ds(s * slab, slab)])
    tail = nn - _NS * slab
    if tail:
        @pl.when(s == 0)
        def _():
            pltpu.sync_copy(src.at[pl.ds(_NS * slab, tail)],
                            dst.at[pl.ds(_NS * slab, tail)])


def _sc_scatter(v, idx2, zc, nch, pw):
    nn = zc.shape[0]
    B = 128
    nb = pw // B
    irw = pw // 128

    @functools.partial(
        pl.kernel, mesh=_mesh(),
        out_type=jax.ShapeDtypeStruct((_NC * nch, nn, 128), jnp.float32),
        scratch_types=(
            [pltpu.VMEM((irw, 128), jnp.int32)]
            + [pltpu.VMEM((B, 128), jnp.float32) for _ in range(2)]
            + [pltpu.VMEM_SHARED((nn, 128), jnp.float32)]
            + [pltpu.SemaphoreType.DMA for _ in range(4)]
        ),
    )
    def kfn(v_hbm, idx_hbm, zc_hbm, agg_hbm, idx_all,
            r0, r1, acc_sh, l0, l1, s0, s1):
        rows = [r0, r1]
        lsem = [l0, l1]
        ssem = [s0, s1]
        c = lax.axis_index("c")
        s = lax.axis_index("s")
        w = s * _NC + c
        base = w * pw
        pltpu.sync_copy(idx_hbm.at[pl.ds(w * irw, irw)], idx_all)

        for ch in range(nch):
            _acc_copy(zc_hbm, acc_sh, s, nn)
            plsc.subcore_barrier()

            def lstart(i, sl):
                pltpu.async_copy(
                    v_hbm.at[pl.ds(base + i * B, B), pl.ds(ch * 128, 128)],
                    rows[sl], lsem[sl])

            def lwait(sl):
                pltpu.make_async_copy(
                    v_hbm.at[pl.ds(base, B), pl.ds(ch * 128, 128)],
                    rows[sl], lsem[sl]).wait()

            def sstart(i, sl):
                pltpu.async_copy(rows[sl], acc_sh.at[idx_all.at[i]],
                                 ssem[sl], add=True)

            def swait(sl):
                pltpu.make_async_copy(
                    v_hbm.at[pl.ds(base, B), pl.ds(ch * 128, 128)],
                    rows[sl], ssem[sl]).wait()

            lstart(0, 0)
            lstart(1, 1)

            @pl.loop(0, nb // 2 - 1)
            def _(t):
                i0 = 2 * t
                lwait(0)
                sstart(i0, 0)
                lwait(1)
                sstart(i0 + 1, 1)
                swait(0)
                lstart(i0 + 2, 0)
                swait(1)
                lstart(i0 + 3, 1)

            lwait(0)
            sstart(nb - 2, 0)
            lwait(1)
            sstart(nb - 1, 1)
            swait(0)
            swait(1)

            plsc.subcore_barrier()
            _acc_copy(acc_sh, agg_hbm.at[c * nch + ch], s, nn)

    return kfn(v, idx2, zc)


def _gat_layer(x, src, dst, dst2, e2, Wl, bl, Wr, br, att, bias, heads, ch):
    n = x.shape[0]
    hc = heads * ch
    e2p = src.shape[0]
    pw = e2p // _NW
    augw = -(hc + heads) % 128 + hc + heads

    af = att.reshape(hc)
    hrep = jnp.repeat(jnp.arange(heads), ch)
    if heads == 8:
        a2_proj = jnp.zeros((hc, 128), jnp.float32).at[
            jnp.arange(hc), hrep].set(af)
        mats = [a2_proj[:256], a2_proj[256:],
                jnp.repeat(jnp.eye(heads, dtype=jnp.float32), ch, axis=1)[:, :256],
                jnp.repeat(jnp.eye(heads, dtype=jnp.float32), ch, axis=1)[:, 256:],
                jnp.eye(heads, 128, dtype=jnp.float32)]
        rpad = None
        xw, ww = 256, 384
    else:
        a2_proj = jnp.zeros((hc, 128), jnp.float32).at[
            jnp.arange(hc), 64].set(af)
        a2_edge = jnp.zeros((hc, 128), jnp.float32).at[
            jnp.arange(hc), 0].set(af)
        mats = [a2_edge,
                (jnp.arange(128) < 64).astype(jnp.float32).reshape(1, 128),
                jnp.zeros((1, 128), jnp.float32).at[0, 64].set(1.0)]
        rpad = jnp.eye(hc, 128, dtype=jnp.float32)
        xw, ww = 128, 128

    xl_t, xr_aug, xr = _dense_proj(x, Wl, bl, Wr, br, a2_proj, rpad, xw, ww)

    bg = 64 if ww > 256 else 128
    xl_src = _sc_gather(xl_t, src, bg, pw)
    xr_g = _sc_gather(xr_aug, dst, bg, pw)

    v_aug = _edge_math(xl_src, xr_g, mats, e2, heads, augw)

    nch = augw // 128
    zc = jnp.zeros((n, 128), jnp.float32)
    parts = _sc_scatter(v_aug, dst2, zc, nch, pw)

    agg = (parts[:nch] + parts[nch:]).transpose(1, 0, 2).reshape(n, nch * 128)
    aggu = agg[:, :hc]
    den = agg[:, hc:hc + heads]
    out = (aggu.reshape(n, heads, ch) / den[..., None]
           - xr.reshape(n, heads, ch)).reshape(n, hc)
    return out + bias


def _bn_gelu(x, g, b):
    mu = x.mean(0)
    var = x.var(0)
    return jax.nn.gelu(g * (x - mu) * jax.lax.rsqrt(var + 1e-5) + b,
                       approximate=False)


def kernel(x, edge_index, Wl1, bl1, Wr1, br1, att1, bias1, g1, be1,
           Wl2, bl2, Wr2, br2, att2, bias2, g2, be2, Wc, bc):
    n = x.shape[0]
    e = edge_index.shape[1]
    e2 = e + n
    grain = _NW * _GRAIN
    e2p = ((e2 + grain - 1) // grain) * grain

    loop = jnp.arange(n, dtype=edge_index.dtype)
    pad = e2p - e2
    src = jnp.pad(jnp.concatenate([edge_index[0], loop]), (0, pad))
    dst = jnp.pad(jnp.concatenate([edge_index[1], loop]), (0, pad))
    dst2 = dst.reshape(e2p // 128, 128)

    h = _gat_layer(x, src, dst, dst2, e2, Wl1, bl1, Wr1, br1, att1, bias1, 8, 64)
    h = _bn_gelu(h, g1, be1)
    h = _gat_layer(h, src, dst, dst2, e2, Wl2, bl2, Wr2, br2, att2, bias2, 1, 64)
    h = _bn_gelu(h, g2, be2)
    return jax.nn.log_softmax(h @ Wc + bc, axis=1)
